# baseline (speedup 1.0000x reference)
import functools

import jax
import jax.numpy as jnp
from jax import lax
from jax.experimental import pallas as pl
from jax.experimental.pallas import tpu as pltpu
from jax.experimental.pallas import tpu_sc as plsc

N, C, H, W = 4, 384, 224, 224
P = H * W
NIMG = N * C
NW = 32
IMGS_PER_W = NIMG // NW
W_PER_N = NW // N
NCHUNK = 8
K = P // NCHUNK
L = 16


def _body(gx_hbm, gy_hbm, z_hbm, out_hbm, packed_v, img_v, buf_v, sem):
    wid = lax.axis_index("s") * 2 + lax.axis_index("c")
    n = wid // W_PER_N

    for c in range(NCHUNK):
        pltpu.sync_copy(gx_hbm.at[n, c], buf_v.at[0])
        pltpu.sync_copy(gy_hbm.at[n, c], buf_v.at[1])

        @pl.loop(0, K // L)
        def _pack(i, c=c):
            s = i * L
            gx = buf_v[0, pl.ds(s, L)]
            gy = buf_v[1, pl.ds(s, L)]
            x = ((gx + 1.0) * 0.5) * (W - 1)
            y = ((gy + 1.0) * 0.5) * (H - 1)
            x = jnp.minimum(jnp.maximum(x, 0.0), float(W - 1))
            y = jnp.minimum(jnp.maximum(y, 0.0), float(H - 1))
            x0 = jnp.minimum(x.astype(jnp.int32), W - 2)
            y0 = jnp.minimum(y.astype(jnp.int32), H - 2)
            wx8 = ((x - x0.astype(jnp.float32)) * 255.0 + 0.5).astype(jnp.int32)
            wy8 = ((y - y0.astype(jnp.float32)) * 255.0 + 0.5).astype(jnp.int32)
            idx = y0 * W + x0
            packed_v[pl.ds(c * K + s, L)] = idx | (wx8 << 16) | (wy8 << 24)

    base_img = wid * IMGS_PER_W

    @pl.loop(0, IMGS_PER_W)
    def _image(j):
        img = base_img + j
        pltpu.sync_copy(z_hbm.at[img], img_v)
        for c in range(NCHUNK):
            slot = c % 2
            if c >= 2:
                pltpu.make_async_copy(buf_v.at[slot], out_hbm.at[img, c], sem).wait()

            @pl.loop(0, K // L)
            def _interp(i, c=c, slot=slot):
                s = i * L
                p = packed_v[pl.ds(c * K + s, L)]
                i00 = p & 0xFFFF
                wx = ((p >> 16) & 0xFF).astype(jnp.float32) * (1.0 / 255.0)
                wy = (lax.shift_right_logical(p, 24)).astype(jnp.float32) * (1.0 / 255.0)
                v00 = plsc.load_gather(img_v, [i00])
                v01 = plsc.load_gather(img_v, [i00 + 1])
                v10 = plsc.load_gather(img_v, [i00 + W])
                v11 = plsc.load_gather(img_v, [i00 + (W + 1)])
                r0 = v00 + wx * (v01 - v00)
                r1 = v10 + wx * (v11 - v10)
                buf_v[slot, pl.ds(s, L)] = r0 + wy * (r1 - r0)

            pltpu.async_copy(buf_v.at[slot], out_hbm.at[img, c], sem)
        for slot in range(2):
            pltpu.make_async_copy(buf_v.at[slot], out_hbm.at[img, NCHUNK - 2 + slot], sem).wait()


@jax.jit
def kernel(z, grid):
    gx = grid[..., 0].reshape(N, NCHUNK, K)
    gy = grid[..., 1].reshape(N, NCHUNK, K)
    z_flat = z.reshape(NIMG, P)

    sampler = pl.kernel(
        _body,
        out_type=jax.ShapeDtypeStruct((NIMG, NCHUNK, K), jnp.float32),
        mesh=plsc.VectorSubcoreMesh(core_axis_name="c", subcore_axis_name="s"),
        scratch_types=[
            pltpu.VMEM((P,), jnp.int32),
            pltpu.VMEM((P,), jnp.float32),
            pltpu.VMEM((2, K), jnp.float32),
            pltpu.SemaphoreType.DMA,
        ],
        compiler_params=pltpu.CompilerParams(needs_layout_passes=False),
    )
    out = sampler(gx, gy, z_flat)
    return out.reshape(N, C, H, W)


# speedup vs baseline: 2.1104x; 2.1104x over previous
import functools

import jax
import jax.numpy as jnp
from jax import lax
from jax.experimental import pallas as pl
from jax.experimental.pallas import tpu as pltpu
from jax.experimental.pallas import tpu_sc as plsc

N, C, H, W = 4, 384, 224, 224
P = H * W
NIMG = N * C
NW = 32
IMGS_PER_W = NIMG // NW
W_PER_N = NW // N
NCHUNK = 8
K = P // NCHUNK
L = 16


def _body(gx_hbm, gy_hbm, z_hbm, out_hbm, packed_v, img_v, buf_v, sem):
    wid = lax.axis_index("s") * 2 + lax.axis_index("c")
    n = wid // W_PER_N

    for c in range(NCHUNK):
        pltpu.sync_copy(gx_hbm.at[n, c], buf_v.at[0])
        pltpu.sync_copy(gy_hbm.at[n, c], buf_v.at[1])

        @plsc.parallel_loop(0, K // L, unroll=4)
        def _pack(i, c=c):
            s = i * L
            gx = buf_v[0, pl.ds(s, L)]
            gy = buf_v[1, pl.ds(s, L)]
            x = ((gx + 1.0) * 0.5) * (W - 1)
            y = ((gy + 1.0) * 0.5) * (H - 1)
            x = jnp.minimum(jnp.maximum(x, 0.0), float(W - 1))
            y = jnp.minimum(jnp.maximum(y, 0.0), float(H - 1))
            x0 = jnp.minimum(x.astype(jnp.int32), W - 2)
            y0 = jnp.minimum(y.astype(jnp.int32), H - 2)
            wx8 = ((x - x0.astype(jnp.float32)) * 255.0 + 0.5).astype(jnp.int32)
            wy8 = ((y - y0.astype(jnp.float32)) * 255.0 + 0.5).astype(jnp.int32)
            idx = y0 * W + x0
            packed_v[pl.ds(c * K + s, L)] = idx | (wx8 << 16) | (wy8 << 24)

    base_img = wid * IMGS_PER_W

    @pl.loop(0, IMGS_PER_W)
    def _image(j):
        img = base_img + j
        pltpu.sync_copy(z_hbm.at[img], img_v)
        for c in range(NCHUNK):
            slot = c % 2
            if c >= 2:
                pltpu.make_async_copy(buf_v.at[slot], out_hbm.at[img, c], sem).wait()

            @plsc.parallel_loop(0, K // L, unroll=4)
            def _interp(i, c=c, slot=slot):
                s = i * L
                p = packed_v[pl.ds(c * K + s, L)]
                i00 = p & 0xFFFF
                wx = ((p >> 16) & 0xFF).astype(jnp.float32) * (1.0 / 255.0)
                wy = (lax.shift_right_logical(p, 24)).astype(jnp.float32) * (1.0 / 255.0)
                v00 = plsc.load_gather(img_v, [i00])
                v01 = plsc.load_gather(img_v, [i00 + 1])
                v10 = plsc.load_gather(img_v, [i00 + W])
                v11 = plsc.load_gather(img_v, [i00 + (W + 1)])
                r0 = v00 + wx * (v01 - v00)
                r1 = v10 + wx * (v11 - v10)
                buf_v[slot, pl.ds(s, L)] = r0 + wy * (r1 - r0)

            pltpu.async_copy(buf_v.at[slot], out_hbm.at[img, c], sem)
        for slot in range(2):
            pltpu.make_async_copy(buf_v.at[slot], out_hbm.at[img, NCHUNK - 2 + slot], sem).wait()


@jax.jit
def kernel(z, grid):
    gx = grid[..., 0].reshape(N, NCHUNK, K)
    gy = grid[..., 1].reshape(N, NCHUNK, K)
    z_flat = z.reshape(NIMG, P)

    sampler = pl.kernel(
        _body,
        out_type=jax.ShapeDtypeStruct((NIMG, NCHUNK, K), jnp.float32),
        mesh=plsc.VectorSubcoreMesh(core_axis_name="c", subcore_axis_name="s"),
        scratch_types=[
            pltpu.VMEM((P,), jnp.int32),
            pltpu.VMEM((P,), jnp.float32),
            pltpu.VMEM((2, K), jnp.float32),
            pltpu.SemaphoreType.DMA,
        ],
        compiler_params=pltpu.CompilerParams(needs_layout_passes=False),
    )
    out = sampler(gx, gy, z_flat)
    return out.reshape(N, C, H, W)
